# baseline (speedup 1.0000x reference)
import functools

import jax
import jax.numpy as jnp
from jax import lax
from jax.experimental import pallas as pl
from jax.experimental.pallas import tpu as pltpu
from jax.experimental.pallas import tpu_sc as plsc

D = 128
NC = 2
NS = 16
CHUNK = 80


WINDOW = 8
NBUF = 4


def _make_spmm(n_nodes, n_edges):
    assert n_edges % (NC * NS * WINDOW * CHUNK) == 0
    n_chunks = n_edges // CHUNK
    tile_chunks = n_chunks // (NC * NS)
    assert tile_chunks % NBUF == 0 and tile_chunks % WINDOW == 0
    n_iters = tile_chunks // NBUF
    n_windows = tile_chunks // WINDOW
    rows_per_tile = n_nodes // NS
    mesh = plsc.VectorSubcoreMesh(core_axis_name="c", subcore_axis_name="s")

    @functools.partial(
        pl.kernel,
        mesh=mesh,
        out_type=jax.ShapeDtypeStruct((NC, n_nodes, D), jnp.float32),
        scratch_types=[
            pltpu.VMEM((2, WINDOW, 2, CHUNK), jnp.int32),
            pltpu.VMEM((2, WINDOW, CHUNK), jnp.float32),
            pltpu.VMEM((NBUF, CHUNK, D), jnp.float32),
            pltpu.VMEM_SHARED((n_nodes, D), jnp.float32),
            [pltpu.SemaphoreType.DMA] * NBUF,
            pltpu.SemaphoreType.DMA,
        ],
        compiler_params=pltpu.CompilerParams(use_tc_tiling_on_sc=False),
    )
    def spmm(rc_hbm, v_hbm, x_hbm, out_hbm, w, wv, rows, acc, gsems, wsem):
        c = lax.axis_index("c")
        s = lax.axis_index("s")
        tile = c * NS + s
        ck0 = tile * tile_chunks

        pltpu.sync_copy(rc_hbm.at[pl.ds(ck0, WINDOW)], w.at[0])
        pltpu.sync_copy(v_hbm.at[pl.ds(ck0, WINDOW)], wv.at[0])
        if n_windows > 1:
            pltpu.sync_copy(rc_hbm.at[pl.ds(ck0 + WINDOW, WINDOW)], w.at[1])
            pltpu.sync_copy(v_hbm.at[pl.ds(ck0 + WINDOW, WINDOW)], wv.at[1])

        def zloop(i, carry):
            z = jnp.zeros((16,), jnp.float32)
            for f in range(D // 16):
                rows[0, i, pl.ds(16 * f, 16)] = z
            return carry
        lax.fori_loop(0, CHUNK, zloop, 0)
        r0 = s * rows_per_tile
        nfull = rows_per_tile // CHUNK
        for j in range(nfull):
            pltpu.sync_copy(rows.at[0], acc.at[pl.ds(r0 + j * CHUNK, CHUNK)])
        rem = rows_per_tile - nfull * CHUNK
        if rem:
            pltpu.sync_copy(rows.at[0, pl.ds(0, rem)],
                            acc.at[pl.ds(r0 + nfull * CHUNK, rem)])
        plsc.subcore_barrier()

        def scale(b, ws, kk):
            def group_body(g, c2):
                vv = wv[ws, kk, pl.ds(g * 16, 16)]
                for j in range(16):
                    v = vv[j]
                    for f in range(D // 16):
                        sl = rows[b, g * 16 + j, pl.ds(16 * f, 16)]
                        rows[b, g * 16 + j, pl.ds(16 * f, 16)] = sl * v
                return c2
            lax.fori_loop(0, CHUNK // 16, group_body, 0)

        for b in range(NBUF):
            pltpu.async_copy(x_hbm.at[w.at[0, b, 1]], rows.at[b], gsems[b])

        def iter_body(q, carry):
            m = q // 2
            ws = m % 2
            odd = q % 2

            @pl.when((odd == 0) & (m + 1 < n_windows))
            def _():
                pltpu.async_copy(
                    rc_hbm.at[pl.ds(ck0 + (m + 1) * WINDOW, WINDOW)],
                    w.at[(m + 1) % 2], wsem)
                pltpu.async_copy(
                    v_hbm.at[pl.ds(ck0 + (m + 1) * WINDOW, WINDOW)],
                    wv.at[(m + 1) % 2], wsem)

            @pl.when((odd == 1) & (m + 1 < n_windows))
            def _():
                pltpu.make_async_copy(
                    rc_hbm.at[pl.ds(ck0 + (m + 1) * WINDOW, WINDOW)],
                    w.at[(m + 1) % 2], wsem).wait()
                pltpu.make_async_copy(
                    v_hbm.at[pl.ds(ck0 + (m + 1) * WINDOW, WINDOW)],
                    wv.at[(m + 1) % 2], wsem).wait()

            for b in range(NBUF):
                ck = NBUF * q + b
                kk = odd * NBUF + b
                pltpu.make_async_copy(x_hbm.at[w.at[ws, kk, 1]],
                                      rows.at[b], gsems[b]).wait()
                scale(b, ws, kk)
                pltpu.sync_copy(rows.at[b], acc.at[w.at[ws, kk, 0]],
                                add=True)

                @pl.when(ck + NBUF < tile_chunks)
                def _():
                    ws_n = jnp.where(odd == 1, (m + 1) % 2, ws)
                    kk_n = jnp.where(odd == 1, b, NBUF + b)
                    pltpu.async_copy(x_hbm.at[w.at[ws_n, kk_n, 1]],
                                     rows.at[b], gsems[b])
            return carry
        lax.fori_loop(0, n_iters, iter_body, 0)
        plsc.subcore_barrier()

        pltpu.sync_copy(acc.at[pl.ds(r0, rows_per_tile)],
                        out_hbm.at[c, pl.ds(r0, rows_per_tile)])

    return spmm


def _layer_tc(x, p0, p1, s_in, w2, wt2):
    n_nodes = x.shape[0]
    nb = 10
    br = n_nodes // nb

    def body(x_ref, p0_ref, p1_ref, s_ref, w_ref, wt_ref, y_ref, so_ref):
        xb = x_ref[...]
        logits = jnp.dot(xb, w_ref[0], preferred_element_type=jnp.float32)
        m = jnp.max(logits, axis=1, keepdims=True)
        e = jnp.exp(logits - m)
        probs = e / jnp.sum(e, axis=1, keepdims=True)
        intent = jnp.dot(probs, wt_ref[0], preferred_element_type=jnp.float32)
        y = xb + p0_ref[...] + p1_ref[...] + intent
        y_ref[...] = y
        so_ref[...] = s_ref[...] + y

    half = nb // 2
    wmap = lambda i: (jnp.where(i < half, 0, 1), 0, 0)
    blk = lambda i: (i, 0)
    return pl.pallas_call(
        body,
        grid=(nb,),
        in_specs=[
            pl.BlockSpec((br, D), blk),
            pl.BlockSpec((br, D), blk),
            pl.BlockSpec((br, D), blk),
            pl.BlockSpec((br, D), blk),
            pl.BlockSpec((1, D, D), wmap),
            pl.BlockSpec((1, D, D), wmap),
        ],
        out_specs=[pl.BlockSpec((br, D), blk), pl.BlockSpec((br, D), blk)],
        out_shape=[jax.ShapeDtypeStruct((n_nodes, D), jnp.float32)] * 2,
    )(x, p0, p1, s_in, w2, wt2)


def kernel(G_indices, G_values, user_emb, item_emb, user_intent, item_intent):
    n_users = user_emb.shape[0]
    n_items = item_emb.shape[0]
    n_nodes = n_users + n_items
    n_edges = G_values.shape[0]

    x0 = jnp.concatenate([user_emb, item_emb], axis=0)
    unit = NC * NS * WINDOW * CHUNK
    n_pad = -(-n_edges // unit) * unit
    row1 = G_indices[0]
    col1 = G_indices[1]
    val1 = G_values
    if n_pad != n_edges:
        pz = n_pad - n_edges
        zi = jnp.zeros((pz,), jnp.int32)
        row1 = jnp.concatenate([row1, zi])
        col1 = jnp.concatenate([col1, zi])
        val1 = jnp.concatenate([val1, jnp.zeros((pz,), jnp.float32)])
    n_chunks = n_pad // CHUNK
    rc = jnp.stack([
        row1.reshape(n_chunks, CHUNK),
        col1.reshape(n_chunks, CHUNK),
    ], axis=1)
    val = val1.reshape(n_chunks, CHUNK)
    w2 = jnp.stack([user_intent, item_intent])
    wt2 = jnp.stack([user_intent.T, item_intent.T])

    spmm = _make_spmm(n_nodes, n_pad)

    p = spmm(rc, val, x0)
    x1, s1 = _layer_tc(x0, p[0], p[1], x0, w2, wt2)
    p2 = spmm(rc, val, x1)
    _, total = _layer_tc(x1, p2[0], p2[1], s1, w2, wt2)

    return total[:n_users], total[n_users:]


# speedup vs baseline: 1.7158x; 1.7158x over previous
import functools

import jax
import jax.numpy as jnp
from jax import lax
from jax.experimental import pallas as pl
from jax.experimental.pallas import tpu as pltpu
from jax.experimental.pallas import tpu_sc as plsc

D = 128
NC = 2
NS = 16
CHUNK = 80


def _make_spmm(n_nodes, n_edges):
    assert n_edges % (NC * NS * 2 * CHUNK) == 0
    n_chunks = n_edges // CHUNK
    tile_chunks = n_chunks // (NC * NS)
    rows_per_tile = n_nodes // NS
    mesh = plsc.VectorSubcoreMesh(core_axis_name="c", subcore_axis_name="s")

    @functools.partial(
        pl.kernel,
        mesh=mesh,
        out_type=jax.ShapeDtypeStruct((NC, n_nodes, D), jnp.float32),
        scratch_types=[
            pltpu.VMEM((tile_chunks, CHUNK), jnp.int32),
            pltpu.VMEM((tile_chunks, CHUNK), jnp.int32),
            pltpu.VMEM((tile_chunks, CHUNK), jnp.float32),
            pltpu.VMEM((CHUNK, D), jnp.float32),
            pltpu.VMEM((CHUNK, D), jnp.float32),
            pltpu.VMEM_SHARED((n_nodes, D), jnp.float32),
            pltpu.SemaphoreType.DMA,
            pltpu.SemaphoreType.DMA,
            pltpu.SemaphoreType.DMA,
            pltpu.SemaphoreType.DMA,
        ],
        compiler_params=pltpu.CompilerParams(use_tc_tiling_on_sc=False),
    )
    def spmm(row_hbm, col_hbm, val_hbm, x_hbm, out_hbm,
             row_vm, col_vm, val_vm, rows0, rows1, acc,
             gsem0, gsem1, ssem0, ssem1):
        c = lax.axis_index("c")
        s = lax.axis_index("s")
        tile = c * NS + s

        ck0 = tile * tile_chunks
        pltpu.sync_copy(row_hbm.at[pl.ds(ck0, tile_chunks)], row_vm)
        pltpu.sync_copy(col_hbm.at[pl.ds(ck0, tile_chunks)], col_vm)
        pltpu.sync_copy(val_hbm.at[pl.ds(ck0, tile_chunks)], val_vm)

        def zloop(i, carry):
            z = jnp.zeros((16,), jnp.float32)
            for f in range(D // 16):
                rows0[i, pl.ds(16 * f, 16)] = z
            return carry
        lax.fori_loop(0, CHUNK, zloop, 0)
        r0 = s * rows_per_tile
        nfull = rows_per_tile // CHUNK
        for j in range(nfull):
            pltpu.sync_copy(rows0, acc.at[pl.ds(r0 + j * CHUNK, CHUNK)])
        rem = rows_per_tile - nfull * CHUNK
        if rem:
            pltpu.sync_copy(rows0.at[pl.ds(0, rem)],
                            acc.at[pl.ds(r0 + nfull * CHUNK, rem)])
        plsc.subcore_barrier()

        def scale(buf, ck):
            def group_body(g, c2):
                vv = val_vm[ck, pl.ds(g * 16, 16)]
                for j in range(16):
                    v = vv[j]
                    for f in range(D // 16):
                        sl = buf[g * 16 + j, pl.ds(16 * f, 16)]
                        buf[g * 16 + j, pl.ds(16 * f, 16)] = sl * v
                return c2
            lax.fori_loop(0, CHUNK // 16, group_body, 0)

        n_pairs = tile_chunks // 2
        pltpu.async_copy(x_hbm.at[col_vm.at[0]], rows0, gsem0)

        def pair_body(p, carry):
            ck = 2 * p
            pltpu.async_copy(x_hbm.at[col_vm.at[ck + 1]], rows1, gsem1)
            pltpu.make_async_copy(x_hbm.at[col_vm.at[ck]], rows0,
                                  gsem0).wait()
            scale(rows0, ck)
            pltpu.sync_copy(rows0, acc.at[row_vm.at[ck]], add=True)

            @pl.when(p + 1 < n_pairs)
            def _():
                pltpu.async_copy(x_hbm.at[col_vm.at[ck + 2]], rows0, gsem0)
            pltpu.make_async_copy(x_hbm.at[col_vm.at[ck + 1]], rows1,
                                  gsem1).wait()
            scale(rows1, ck + 1)
            pltpu.sync_copy(rows1, acc.at[row_vm.at[ck + 1]], add=True)
            return carry
        lax.fori_loop(0, n_pairs, pair_body, 0)
        plsc.subcore_barrier()

        pltpu.sync_copy(acc.at[pl.ds(r0, rows_per_tile)],
                        out_hbm.at[c, pl.ds(r0, rows_per_tile)])

    return spmm


def _layer_tc(x, p0, p1, s_in, w2, wt2):
    n_nodes = x.shape[0]
    nb = 10
    br = n_nodes // nb

    def body(x_ref, p0_ref, p1_ref, s_ref, w_ref, wt_ref, y_ref, so_ref):
        xb = x_ref[...]
        logits = jnp.dot(xb, w_ref[0], preferred_element_type=jnp.float32)
        m = jnp.max(logits, axis=1, keepdims=True)
        e = jnp.exp(logits - m)
        probs = e / jnp.sum(e, axis=1, keepdims=True)
        intent = jnp.dot(probs, wt_ref[0], preferred_element_type=jnp.float32)
        y = xb + p0_ref[...] + p1_ref[...] + intent
        y_ref[...] = y
        so_ref[...] = s_ref[...] + y

    half = nb // 2
    wmap = lambda i: (jnp.where(i < half, 0, 1), 0, 0)
    blk = lambda i: (i, 0)
    return pl.pallas_call(
        body,
        grid=(nb,),
        in_specs=[
            pl.BlockSpec((br, D), blk),
            pl.BlockSpec((br, D), blk),
            pl.BlockSpec((br, D), blk),
            pl.BlockSpec((br, D), blk),
            pl.BlockSpec((1, D, D), wmap),
            pl.BlockSpec((1, D, D), wmap),
        ],
        out_specs=[pl.BlockSpec((br, D), blk), pl.BlockSpec((br, D), blk)],
        out_shape=[jax.ShapeDtypeStruct((n_nodes, D), jnp.float32)] * 2,
    )(x, p0, p1, s_in, w2, wt2)


def kernel(G_indices, G_values, user_emb, item_emb, user_intent, item_intent):
    n_users = user_emb.shape[0]
    n_items = item_emb.shape[0]
    n_nodes = n_users + n_items
    n_edges = G_values.shape[0]

    x0 = jnp.concatenate([user_emb, item_emb], axis=0)
    unit = NC * NS * 2 * CHUNK
    n_pad = -(-n_edges // unit) * unit
    row1 = G_indices[0]
    col1 = G_indices[1]
    val1 = G_values
    if n_pad != n_edges:
        pz = n_pad - n_edges
        zi = jnp.zeros((pz,), jnp.int32)
        row1 = jnp.concatenate([row1, zi])
        col1 = jnp.concatenate([col1, zi])
        val1 = jnp.concatenate([val1, jnp.zeros((pz,), jnp.float32)])
    n_chunks = n_pad // CHUNK
    row = row1.reshape(n_chunks, CHUNK)
    col = col1.reshape(n_chunks, CHUNK)
    val = val1.reshape(n_chunks, CHUNK)
    w2 = jnp.stack([user_intent, item_intent])
    wt2 = jnp.stack([user_intent.T, item_intent.T])

    spmm = _make_spmm(n_nodes, n_pad)

    p = spmm(row, col, val, x0)
    x1, s1 = _layer_tc(x0, p[0], p[1], x0, w2, wt2)
    p2 = spmm(row, col, val, x1)
    _, total = _layer_tc(x1, p2[0], p2[1], s1, w2, wt2)

    return total[:n_users], total[n_users:]


# speedup vs baseline: 1.7346x; 1.0109x over previous
import functools

import jax
import jax.numpy as jnp
from jax import lax
from jax.experimental import pallas as pl
from jax.experimental.pallas import tpu as pltpu
from jax.experimental.pallas import tpu_sc as plsc

D = 128
NC = 2
NS = 16
CHUNK = 80


def _make_spmm(n_nodes, n_edges):
    assert n_edges % (NC * NS * 2 * CHUNK) == 0
    n_chunks = n_edges // CHUNK
    tile_chunks = n_chunks // (NC * NS)
    rows_per_tile = n_nodes // NS
    mesh = plsc.VectorSubcoreMesh(core_axis_name="c", subcore_axis_name="s")

    @functools.partial(
        pl.kernel,
        mesh=mesh,
        out_type=jax.ShapeDtypeStruct((NC, n_nodes, D), jnp.float32),
        scratch_types=[
            pltpu.VMEM((tile_chunks, CHUNK), jnp.int32),
            pltpu.VMEM((tile_chunks, CHUNK), jnp.int32),
            pltpu.VMEM((tile_chunks, CHUNK), jnp.float32),
            pltpu.VMEM((CHUNK, D), jnp.float32),
            pltpu.VMEM((CHUNK, D), jnp.float32),
            pltpu.VMEM_SHARED((n_nodes, D), jnp.float32),
            pltpu.SemaphoreType.DMA,
            pltpu.SemaphoreType.DMA,
            pltpu.SemaphoreType.DMA,
            pltpu.SemaphoreType.DMA,
        ],
        compiler_params=pltpu.CompilerParams(use_tc_tiling_on_sc=False),
    )
    def spmm(row_hbm, col_hbm, val_hbm, x_hbm, out_hbm,
             row_vm, col_vm, val_vm, rows0, rows1, acc,
             gsem0, gsem1, ssem0, ssem1):
        c = lax.axis_index("c")
        s = lax.axis_index("s")
        tile = c * NS + s

        ck0 = tile * tile_chunks
        pltpu.async_copy(row_hbm.at[pl.ds(ck0, tile_chunks)], row_vm, ssem0)
        pltpu.async_copy(col_hbm.at[pl.ds(ck0, tile_chunks)], col_vm, ssem1)
        pltpu.async_copy(val_hbm.at[pl.ds(ck0, tile_chunks)], val_vm, gsem1)

        def zloop(i, carry):
            z = jnp.zeros((16,), jnp.float32)
            for f in range(D // 16):
                rows0[i, pl.ds(16 * f, 16)] = z
            return carry
        lax.fori_loop(0, CHUNK, zloop, 0)
        r0 = s * rows_per_tile
        nfull = rows_per_tile // CHUNK
        for j in range(nfull):
            pltpu.async_copy(rows0, acc.at[pl.ds(r0 + j * CHUNK, CHUNK)],
                             gsem0)
        rem = rows_per_tile - nfull * CHUNK
        if rem:
            pltpu.async_copy(rows0.at[pl.ds(0, rem)],
                             acc.at[pl.ds(r0 + nfull * CHUNK, rem)], gsem0)
        for j in range(nfull):
            pltpu.make_async_copy(rows0, acc.at[pl.ds(r0 + j * CHUNK, CHUNK)],
                                  gsem0).wait()
        if rem:
            pltpu.make_async_copy(rows0.at[pl.ds(0, rem)],
                                  acc.at[pl.ds(r0 + nfull * CHUNK, rem)],
                                  gsem0).wait()
        pltpu.make_async_copy(row_hbm.at[pl.ds(ck0, tile_chunks)], row_vm,
                              ssem0).wait()
        pltpu.make_async_copy(col_hbm.at[pl.ds(ck0, tile_chunks)], col_vm,
                              ssem1).wait()
        pltpu.make_async_copy(val_hbm.at[pl.ds(ck0, tile_chunks)], val_vm,
                              gsem1).wait()
        plsc.subcore_barrier()

        def scale(buf, ck):
            def group_body(g, c2):
                vv = val_vm[ck, pl.ds(g * 16, 16)]
                for j in range(16):
                    v = vv[j]
                    for f in range(D // 16):
                        sl = buf[g * 16 + j, pl.ds(16 * f, 16)]
                        buf[g * 16 + j, pl.ds(16 * f, 16)] = sl * v
                return c2
            lax.fori_loop(0, CHUNK // 16, group_body, 0)

        n_pairs = tile_chunks // 2
        pltpu.async_copy(x_hbm.at[col_vm.at[0]], rows0, gsem0)

        def pair_body(p, carry):
            ck = 2 * p
            pltpu.async_copy(x_hbm.at[col_vm.at[ck + 1]], rows1, gsem1)
            pltpu.make_async_copy(x_hbm.at[col_vm.at[ck]], rows0,
                                  gsem0).wait()
            scale(rows0, ck)
            pltpu.sync_copy(rows0, acc.at[row_vm.at[ck]], add=True)

            @pl.when(p + 1 < n_pairs)
            def _():
                pltpu.async_copy(x_hbm.at[col_vm.at[ck + 2]], rows0, gsem0)
            pltpu.make_async_copy(x_hbm.at[col_vm.at[ck + 1]], rows1,
                                  gsem1).wait()
            scale(rows1, ck + 1)
            pltpu.sync_copy(rows1, acc.at[row_vm.at[ck + 1]], add=True)
            return carry
        lax.fori_loop(0, n_pairs, pair_body, 0)
        plsc.subcore_barrier()

        pltpu.sync_copy(acc.at[pl.ds(r0, rows_per_tile)],
                        out_hbm.at[c, pl.ds(r0, rows_per_tile)])

    return spmm


def _layer_tc(x, p0, p1, s_in, w2, wt2):
    n_nodes = x.shape[0]
    nb = 10
    br = n_nodes // nb

    def body(x_ref, p0_ref, p1_ref, s_ref, w_ref, wt_ref, y_ref, so_ref):
        xb = x_ref[...]
        logits = jnp.dot(xb, w_ref[0], preferred_element_type=jnp.float32)
        m = jnp.max(logits, axis=1, keepdims=True)
        e = jnp.exp(logits - m)
        probs = e / jnp.sum(e, axis=1, keepdims=True)
        intent = jnp.dot(probs, wt_ref[0], preferred_element_type=jnp.float32)
        y = xb + p0_ref[...] + p1_ref[...] + intent
        y_ref[...] = y
        so_ref[...] = s_ref[...] + y

    half = nb // 2
    wmap = lambda i: (jnp.where(i < half, 0, 1), 0, 0)
    blk = lambda i: (i, 0)
    return pl.pallas_call(
        body,
        grid=(nb,),
        in_specs=[
            pl.BlockSpec((br, D), blk),
            pl.BlockSpec((br, D), blk),
            pl.BlockSpec((br, D), blk),
            pl.BlockSpec((br, D), blk),
            pl.BlockSpec((1, D, D), wmap),
            pl.BlockSpec((1, D, D), wmap),
        ],
        out_specs=[pl.BlockSpec((br, D), blk), pl.BlockSpec((br, D), blk)],
        out_shape=[jax.ShapeDtypeStruct((n_nodes, D), jnp.float32)] * 2,
    )(x, p0, p1, s_in, w2, wt2)


def kernel(G_indices, G_values, user_emb, item_emb, user_intent, item_intent):
    n_users = user_emb.shape[0]
    n_items = item_emb.shape[0]
    n_nodes = n_users + n_items
    n_edges = G_values.shape[0]

    x0 = jnp.concatenate([user_emb, item_emb], axis=0)
    unit = NC * NS * 2 * CHUNK
    n_pad = -(-n_edges // unit) * unit
    row1 = G_indices[0]
    col1 = G_indices[1]
    val1 = G_values
    if n_pad != n_edges:
        pz = n_pad - n_edges
        zi = jnp.zeros((pz,), jnp.int32)
        row1 = jnp.concatenate([row1, zi])
        col1 = jnp.concatenate([col1, zi])
        val1 = jnp.concatenate([val1, jnp.zeros((pz,), jnp.float32)])
    n_chunks = n_pad // CHUNK
    row = row1.reshape(n_chunks, CHUNK)
    col = col1.reshape(n_chunks, CHUNK)
    val = val1.reshape(n_chunks, CHUNK)
    w2 = jnp.stack([user_intent, item_intent])
    wt2 = jnp.stack([user_intent.T, item_intent.T])

    spmm = _make_spmm(n_nodes, n_pad)

    p = spmm(row, col, val, x0)
    x1, s1 = _layer_tc(x0, p[0], p[1], x0, w2, wt2)
    p2 = spmm(row, col, val, x1)
    _, total = _layer_tc(x1, p2[0], p2[1], s1, w2, wt2)

    return total[:n_users], total[n_users:]
